# baseline (speedup 1.0000x reference)
import functools

import jax
import jax.numpy as jnp
from jax import lax
from jax.experimental import pallas as pl
from jax.experimental.pallas import tpu as pltpu
from jax.experimental.pallas import tpu_sc as plsc

S_LEN = 8192
BATCH = 4
D_MODEL = 1024

_info = plsc.get_sparse_core_info()
_NC, _NS = _info.num_cores, _info.num_subcores
_NW = _NC * _NS

_ROWS_PER_W = S_LEN // _NW
_CH = 32
_NCHUNK = _ROWS_PER_W // _CH


_NBUF = 3


def _body(emb_hbm, out_hbm, buf0, buf1, buf2, rs0, rs1, rs2, ws0, ws1, ws2):
    bufs, rsems, wsems = (buf0, buf1, buf2), (rs0, rs1, rs2), (ws0, ws1, ws2)
    wid = lax.axis_index("s") * _NC + lax.axis_index("c")
    base = wid * _ROWS_PER_W

    def read(g, sl):
        return pltpu.async_copy(
            emb_hbm.at[pl.ds(base + g * _CH, _CH)], bufs[sl], rsems[sl])

    rh = [read(g, g) for g in range(_NBUF)]
    wh = [None] * _NBUF
    for g in range(_NCHUNK):
        sl = g % _NBUF
        rh[sl].wait()
        b = base + g * _CH
        wh[sl] = [
            pltpu.async_copy(bufs[sl], out_hbm.at[pl.ds(b, _CH), n], wsems[sl])
            for n in range(BATCH)
        ]
        p = g - 1
        if p >= 0 and p + _NBUF < _NCHUNK:
            psl = p % _NBUF
            for h in wh[psl]:
                h.wait()
            rh[psl] = read(p + _NBUF, psl)
    for sl in range(_NBUF):
        for h in wh[sl]:
            h.wait()


@jax.jit
def _pos_broadcast(pos_embedding):
    mesh = plsc.VectorSubcoreMesh(core_axis_name="c", subcore_axis_name="s")
    return pl.kernel(
        _body,
        out_type=jax.ShapeDtypeStruct((S_LEN, BATCH, D_MODEL), jnp.float32),
        mesh=mesh,
        scratch_types=(
            [pltpu.VMEM((_CH, D_MODEL), jnp.float32)] * _NBUF
            + [pltpu.SemaphoreType.DMA] * (2 * _NBUF)
        ),
    )(pos_embedding)


def kernel(x, pos_embedding):
    del x
    return _pos_broadcast(pos_embedding)
